# baseline (speedup 1.0000x reference)
import jax
import jax.numpy as jnp
from jax import lax
from jax.experimental import pallas as pl
from jax.experimental.pallas import tpu as pltpu
from jax.experimental.pallas import tpu_sc as plsc

NC = 2
NS = 16
L = 16
B = 80
C = 2000
R = 400


def _sc_mesh():
    return plsc.VectorSubcoreMesh(
        core_axis_name="c", subcore_axis_name="s", num_cores=NC, num_subcores=NS
    )


def _make_deg(n, e):
    accrows = n + 8
    srows = (accrows // NS) // 8 * 8
    tail = accrows - NS * srows
    wrows = (n // NS) // 8 * 8
    wtail = n - NS * wrows
    e_tile = e // (NC * NS)
    e_pad = (e_tile + B - 1) // B * B
    nb = e_pad // B

    npadv = (e_pad - (e_tile // L) * L) // L

    def body(dst_hbm, init_hbm, deg_out, didxb, ones80, acc, sem):
        c = lax.axis_index("c")
        s = lax.axis_index("s")
        pltpu.sync_copy(init_hbm.at[pl.ds(0, srows)],
                        acc.at[pl.ds(s * srows, srows)])

        @pl.when(s == 0)
        def _():
            pltpu.sync_copy(init_hbm.at[pl.ds(0, tail)],
                            acc.at[pl.ds(NS * srows, tail)])

        pltpu.sync_copy(init_hbm.at[pl.ds(srows, B)], ones80)
        for j in range(npadv):
            didxb[pl.ds(e_pad - (j + 1) * L, L)] = jnp.full((L,), n, jnp.int32)
        pltpu.sync_copy(dst_hbm.at[pl.ds((c * NS + s) * e_tile, e_tile)],
                        didxb.at[pl.ds(0, e_tile)])
        plsc.subcore_barrier()

        def batch(b, _):
            pltpu.async_copy(ones80, acc.at[didxb.at[pl.ds(b * B, B)]],
                             sem, add=True)
            return 0

        lax.fori_loop(0, nb, batch, 0)

        def drain(b, _):
            pltpu.make_async_copy(ones80, acc.at[didxb.at[pl.ds(b * B, B)]],
                                  sem).wait()
            return 0

        lax.fori_loop(0, nb, drain, 0)
        plsc.subcore_barrier()
        pltpu.sync_copy(acc.at[pl.ds(s * wrows, wrows)],
                        deg_out.at[c, pl.ds(s * wrows, wrows)])

        @pl.when(s == 0)
        def _():
            pltpu.sync_copy(acc.at[pl.ds(NS * wrows, wtail)],
                            deg_out.at[c, pl.ds(NS * wrows, wtail)])

    def run(dst):
        init_hbm = jnp.concatenate(
            [jnp.zeros((srows, L), jnp.float32),
             jnp.ones((B, L), jnp.float32)])
        return pl.kernel(
            body,
            out_type=jax.ShapeDtypeStruct((NC, n, L), jnp.float32),
            mesh=_sc_mesh(),
            scratch_types=[
                pltpu.VMEM((e_pad,), jnp.int32),
                pltpu.VMEM((B, L), jnp.float32),
                pltpu.VMEM_SHARED((accrows, L), jnp.float32),
                pltpu.SemaphoreType.DMA,
            ],
        )(dst, init_hbm)

    return run


def _make_agg(n, e):
    d2 = 128
    srows = (n // NS) // 8 * 8
    tail = n - NS * srows
    e_tile = e // NS
    nch = e_tile // C
    cb = C // B
    ntri = (cb - 4) // 3

    def body(xn_hbm, src_hbm, dst_hbm, out_hbm,
             sidx, didx, r0, r1, acc, sg0, sg1, ss0, ss1):
        c = lax.axis_index("c")
        s = lax.axis_index("s")
        roff = c * n
        ebase = s * e_tile
        pltpu.sync_copy(xn_hbm.at[pl.ds(roff + s * srows, srows)],
                        acc.at[pl.ds(s * srows, srows)])

        @pl.when(s == 0)
        def _():
            pltpu.sync_copy(xn_hbm.at[pl.ds(roff + NS * srows, tail)],
                            acc.at[pl.ds(NS * srows, tail)])

        plsc.subcore_barrier()

        def fire_g(b, rows, sem):
            pltpu.async_copy(xn_hbm.at[sidx.at[pl.ds(b * B, B)]], rows, sem)

        def wait_g(b, rows, sem):
            pltpu.make_async_copy(xn_hbm.at[sidx.at[pl.ds(b * B, B)]],
                                  rows, sem).wait()

        def fire_s(b, rows, sem):
            pltpu.async_copy(rows, acc.at[didx.at[pl.ds(b * B, B)]], sem,
                             add=True)

        def wait_s(b, rows, sem):
            pltpu.make_async_copy(rows, acc.at[didx.at[pl.ds(b * B, B)]],
                                  sem).wait()

        for k in range(nch):
            pltpu.sync_copy(src_hbm.at[pl.ds(ebase + k * C, C)], sidx)
            pltpu.sync_copy(dst_hbm.at[pl.ds(ebase + k * C, C)], didx)

            def adjust(i, _):
                sidx[pl.ds(i * L, L)] = sidx[pl.ds(i * L, L)] + roff
                return 0

            lax.fori_loop(0, C // L, adjust, 0)

            fire_g(0, r0, sg0)

            def pair(p, _):
                b = 2 * p
                fire_g(b + 1, r1, sg1)
                wait_g(b, r0, sg0)
                fire_s(b, r0, ss0)
                wait_s(b, r0, ss0)
                fire_g(b + 2, r0, sg0)
                wait_g(b + 1, r1, sg1)
                fire_s(b + 1, r1, ss1)
                wait_s(b + 1, r1, ss1)
                return 0

            lax.fori_loop(0, (cb - 1) // 2, pair, 0)
            wait_g(cb - 1, r0, sg0)
            fire_s(cb - 1, r0, ss0)
            wait_s(cb - 1, r0, ss0)

        plsc.subcore_barrier()
        pltpu.sync_copy(acc.at[pl.ds(s * srows, srows)],
                        out_hbm.at[c, pl.ds(s * srows, srows)])

        @pl.when(s == 0)
        def _():
            pltpu.sync_copy(acc.at[pl.ds(NS * srows, tail)],
                            out_hbm.at[c, pl.ds(NS * srows, tail)])

    return pl.kernel(
        body,
        out_type=jax.ShapeDtypeStruct((NC, n, d2), jnp.float32),
        mesh=_sc_mesh(),
        scratch_types=[
            pltpu.VMEM((C,), jnp.int32),
            pltpu.VMEM((C,), jnp.int32),
            pltpu.VMEM((B, d2), jnp.float32),
            pltpu.VMEM((B, d2), jnp.float32),
            pltpu.VMEM_SHARED((n, d2), jnp.float32),
            pltpu.SemaphoreType.DMA,
            pltpu.SemaphoreType.DMA,
            pltpu.SemaphoreType.DMA,
            pltpu.SemaphoreType.DMA,
        ],
    )


def _mm0_body(x_ref, w_ref, h_ref):
    h = jnp.dot(x_ref[...], w_ref[...], preferred_element_type=jnp.float32)
    h_ref[0] = h[:, :128]
    h_ref[1] = h[:, 128:]


def _mm0(x, w):
    n, d = x.shape
    h = w.shape[1]
    g = n // R
    return pl.pallas_call(
        _mm0_body,
        grid=(g,),
        in_specs=[
            pl.BlockSpec((R, d), lambda i: (i, 0)),
            pl.BlockSpec((d, h), lambda i: (0, 0)),
        ],
        out_specs=pl.BlockSpec((NC, R, h // 2), lambda i: (0, i, 0)),
        out_shape=jax.ShapeDtypeStruct((NC, n, h // 2), jnp.float32),
    )(x, w)


def _scale_body(h_ref, deg_ref, xn_ref, dinv_ref):
    dinv = lax.rsqrt(deg_ref[0] + deg_ref[1] + 1.0)
    d1 = dinv[:, :1]
    xn_ref[0] = h_ref[0] * d1
    xn_ref[1] = h_ref[1] * d1
    dinv_ref[...] = dinv


def _scale(h, deg):
    n = h.shape[1]
    h2 = h.shape[2]
    g = n // R
    return pl.pallas_call(
        _scale_body,
        grid=(g,),
        in_specs=[
            pl.BlockSpec((NC, R, h2), lambda i: (0, i, 0)),
            pl.BlockSpec((NC, R, L), lambda i: (0, i, 0)),
        ],
        out_specs=[
            pl.BlockSpec((NC, R, h2), lambda i: (0, i, 0)),
            pl.BlockSpec((R, L), lambda i: (i, 0)),
        ],
        out_shape=[
            jax.ShapeDtypeStruct((NC, n, h2), jnp.float32),
            jax.ShapeDtypeStruct((n, L), jnp.float32),
        ],
    )(h, deg)


def _mm2_body(agg_ref, dinv_ref, b_ref, w_ref, xn_ref):
    dinv = dinv_ref[:, :1]
    a = jnp.concatenate([agg_ref[0], agg_ref[1]], axis=1)
    a = jnp.maximum(a * dinv + b_ref[...], 0.0)
    hs = jnp.dot(a, w_ref[...], preferred_element_type=jnp.float32) * dinv
    xn_ref[0] = hs[:, :128]
    xn_ref[1] = hs[:, 128:]


def _mm2(agg, dinv, b, w):
    n = agg.shape[1]
    h = w.shape[0]
    h2 = w.shape[1]
    g = n // R
    return pl.pallas_call(
        _mm2_body,
        grid=(g,),
        in_specs=[
            pl.BlockSpec((NC, R, h // 2), lambda i: (0, i, 0)),
            pl.BlockSpec((R, L), lambda i: (i, 0)),
            pl.BlockSpec((1, h), lambda i: (0, 0)),
            pl.BlockSpec((h, h2), lambda i: (0, 0)),
        ],
        out_specs=pl.BlockSpec((NC, R, h2 // 2), lambda i: (0, i, 0)),
        out_shape=jax.ShapeDtypeStruct((NC, n, h2 // 2), jnp.float32),
    )(agg, dinv, b, w)


def _head_body(agg_ref, dinv_ref, b_ref, wf1_ref, bf1_ref, wf2_ref, bf2_ref,
               out_ref):
    dinv = dinv_ref[:, :1]
    a = jnp.concatenate([agg_ref[0], agg_ref[1]], axis=1)
    a = jnp.maximum(a * dinv + b_ref[...], 0.0)
    f = jnp.maximum(
        jnp.dot(a, wf1_ref[...], preferred_element_type=jnp.float32)
        + bf1_ref[...], 0.0)
    o = (jnp.dot(f, wf2_ref[...], preferred_element_type=jnp.float32)
         + bf2_ref[...])
    m = jnp.max(o, axis=1, keepdims=True)
    z = o - m
    out_ref[...] = z - jnp.log(jnp.sum(jnp.exp(z), axis=1, keepdims=True))


def _head(agg, dinv, b, wf1, bf1, wf2, bf2):
    n = agg.shape[1]
    h = wf1.shape[0]
    o = wf2.shape[1]
    g = n // R
    return pl.pallas_call(
        _head_body,
        grid=(g,),
        in_specs=[
            pl.BlockSpec((NC, R, h // 2), lambda i: (0, i, 0)),
            pl.BlockSpec((R, L), lambda i: (i, 0)),
            pl.BlockSpec((1, h), lambda i: (0, 0)),
            pl.BlockSpec((h, h), lambda i: (0, 0)),
            pl.BlockSpec((1, h), lambda i: (0, 0)),
            pl.BlockSpec((h, o), lambda i: (0, 0)),
            pl.BlockSpec((1, o), lambda i: (0, 0)),
        ],
        out_specs=pl.BlockSpec((R, o), lambda i: (i, 0)),
        out_shape=jax.ShapeDtypeStruct((n, o), jnp.float32),
    )(agg, dinv, b, wf1, bf1, wf2, bf2)


def kernel(x, edge_index, W1, b1, W2, b2, Wf1, bf1, Wf2, bf2):
    n, d = x.shape
    e = edge_index.shape[1]
    src = edge_index[0].astype(jnp.int32)
    dst = edge_index[1].astype(jnp.int32)

    agg = _make_agg(n, e)
    h1 = _mm0(x, W1)
    deg = _make_deg(n, e)(dst)
    xn1, dinv = _scale(h1, deg)
    agg1 = agg(xn1.reshape(NC * n, 128), src, dst)
    xn2 = _mm2(agg1, dinv, b1.reshape(1, -1), W2)
    agg2 = agg(xn2.reshape(NC * n, 128), src, dst)
    return _head(agg2, dinv, b2.reshape(1, -1), Wf1,
                 bf1.reshape(1, -1), Wf2, bf2.reshape(1, -1))


# speedup vs baseline: 1.0009x; 1.0009x over previous
import jax
import jax.numpy as jnp
from jax import lax
from jax.experimental import pallas as pl
from jax.experimental.pallas import tpu as pltpu
from jax.experimental.pallas import tpu_sc as plsc

NC = 2
NS = 16
L = 16
B = 80
C = 2000
R = 400


def _sc_mesh():
    return plsc.VectorSubcoreMesh(
        core_axis_name="c", subcore_axis_name="s", num_cores=NC, num_subcores=NS
    )


def _make_deg(n, e):
    accrows = n + 8
    srows = (accrows // NS) // 8 * 8
    tail = accrows - NS * srows
    wrows = (n // NS) // 8 * 8
    wtail = n - NS * wrows
    e_tile = e // (NC * NS)
    e_pad = (e_tile + B - 1) // B * B
    nb = e_pad // B

    npadv = (e_pad - (e_tile // L) * L) // L

    def body(dst_hbm, init_hbm, deg_out, didxb, ones80, acc, sem):
        c = lax.axis_index("c")
        s = lax.axis_index("s")
        pltpu.sync_copy(init_hbm.at[pl.ds(0, srows)],
                        acc.at[pl.ds(s * srows, srows)])

        @pl.when(s == 0)
        def _():
            pltpu.sync_copy(init_hbm.at[pl.ds(0, tail)],
                            acc.at[pl.ds(NS * srows, tail)])

        pltpu.sync_copy(init_hbm.at[pl.ds(srows, B)], ones80)
        for j in range(npadv):
            didxb[pl.ds(e_pad - (j + 1) * L, L)] = jnp.full((L,), n, jnp.int32)
        pltpu.sync_copy(dst_hbm.at[pl.ds((c * NS + s) * e_tile, e_tile)],
                        didxb.at[pl.ds(0, e_tile)])
        plsc.subcore_barrier()

        def batch(b, _):
            pltpu.async_copy(ones80, acc.at[didxb.at[pl.ds(b * B, B)]],
                             sem, add=True)
            return 0

        lax.fori_loop(0, nb, batch, 0)

        def drain(b, _):
            pltpu.make_async_copy(ones80, acc.at[didxb.at[pl.ds(b * B, B)]],
                                  sem).wait()
            return 0

        lax.fori_loop(0, nb, drain, 0)
        plsc.subcore_barrier()
        pltpu.sync_copy(acc.at[pl.ds(s * wrows, wrows)],
                        deg_out.at[c, pl.ds(s * wrows, wrows)])

        @pl.when(s == 0)
        def _():
            pltpu.sync_copy(acc.at[pl.ds(NS * wrows, wtail)],
                            deg_out.at[c, pl.ds(NS * wrows, wtail)])

    def run(dst):
        init_hbm = jnp.concatenate(
            [jnp.zeros((srows, L), jnp.float32),
             jnp.ones((B, L), jnp.float32)])
        return pl.kernel(
            body,
            out_type=jax.ShapeDtypeStruct((NC, n, L), jnp.float32),
            mesh=_sc_mesh(),
            scratch_types=[
                pltpu.VMEM((e_pad,), jnp.int32),
                pltpu.VMEM((B, L), jnp.float32),
                pltpu.VMEM_SHARED((accrows, L), jnp.float32),
                pltpu.SemaphoreType.DMA,
            ],
        )(dst, init_hbm)

    return run


def _make_agg(n, e):
    d2 = 128
    srows = (n // NS) // 8 * 8
    tail = n - NS * srows
    e_tile = e // NS
    nch = e_tile // C
    cb = C // B

    def body(xn_hbm, src_hbm, dst_hbm, out_hbm,
             sidx, didx, r0, r1, acc, sg0, sg1, ss0, ss1):
        c = lax.axis_index("c")
        s = lax.axis_index("s")
        roff = c * n
        ebase = s * e_tile
        pltpu.sync_copy(xn_hbm.at[pl.ds(roff + s * srows, srows)],
                        acc.at[pl.ds(s * srows, srows)])

        @pl.when(s == 0)
        def _():
            pltpu.sync_copy(xn_hbm.at[pl.ds(roff + NS * srows, tail)],
                            acc.at[pl.ds(NS * srows, tail)])

        plsc.subcore_barrier()

        def fire_g(b, rows, sem):
            pltpu.async_copy(xn_hbm.at[sidx.at[pl.ds(b * B, B)]], rows, sem)

        def wait_g(b, rows, sem):
            pltpu.make_async_copy(xn_hbm.at[sidx.at[pl.ds(b * B, B)]],
                                  rows, sem).wait()

        def fire_s(b, rows, sem):
            pltpu.async_copy(rows, acc.at[didx.at[pl.ds(b * B, B)]], sem,
                             add=True)

        def wait_s(b, rows, sem):
            pltpu.make_async_copy(rows, acc.at[didx.at[pl.ds(b * B, B)]],
                                  sem).wait()

        for k in range(nch):
            pltpu.sync_copy(src_hbm.at[pl.ds(ebase + k * C, C)], sidx)
            pltpu.sync_copy(dst_hbm.at[pl.ds(ebase + k * C, C)], didx)

            def adjust(i, _):
                sidx[pl.ds(i * L, L)] = sidx[pl.ds(i * L, L)] + roff
                return 0

            lax.fori_loop(0, C // L, adjust, 0)

            fire_g(0, r0, sg0)

            def pair(p, _):
                b = 2 * p
                fire_g(b + 1, r1, sg1)
                wait_g(b, r0, sg0)
                fire_s(b, r0, ss0)
                wait_s(b, r0, ss0)
                fire_g(b + 2, r0, sg0)
                wait_g(b + 1, r1, sg1)
                fire_s(b + 1, r1, ss1)
                wait_s(b + 1, r1, ss1)
                return 0

            lax.fori_loop(0, (cb - 1) // 2, pair, 0)
            wait_g(cb - 1, r0, sg0)
            fire_s(cb - 1, r0, ss0)
            wait_s(cb - 1, r0, ss0)

        plsc.subcore_barrier()
        pltpu.sync_copy(acc.at[pl.ds(s * srows, srows)],
                        out_hbm.at[c, pl.ds(s * srows, srows)])

        @pl.when(s == 0)
        def _():
            pltpu.sync_copy(acc.at[pl.ds(NS * srows, tail)],
                            out_hbm.at[c, pl.ds(NS * srows, tail)])

    return pl.kernel(
        body,
        out_type=jax.ShapeDtypeStruct((NC, n, d2), jnp.float32),
        mesh=_sc_mesh(),
        scratch_types=[
            pltpu.VMEM((C,), jnp.int32),
            pltpu.VMEM((C,), jnp.int32),
            pltpu.VMEM((B, d2), jnp.float32),
            pltpu.VMEM((B, d2), jnp.float32),
            pltpu.VMEM_SHARED((n, d2), jnp.float32),
            pltpu.SemaphoreType.DMA,
            pltpu.SemaphoreType.DMA,
            pltpu.SemaphoreType.DMA,
            pltpu.SemaphoreType.DMA,
        ],
    )


def _mm0_body(x_ref, w_ref, h_ref):
    h_ref[...] = jnp.dot(x_ref[...], w_ref[...],
                         preferred_element_type=jnp.float32)


def _mm0(x, w):
    n, d = x.shape
    h = w.shape[1]
    g = n // R
    return pl.pallas_call(
        _mm0_body,
        grid=(g,),
        in_specs=[
            pl.BlockSpec((R, d), lambda i: (i, 0)),
            pl.BlockSpec((d, h), lambda i: (0, 0)),
        ],
        out_specs=pl.BlockSpec((R, h), lambda i: (i, 0)),
        out_shape=jax.ShapeDtypeStruct((n, h), jnp.float32),
    )(x, w)


def _scale_body(h_ref, deg_ref, xn_ref, dinv_ref):
    dinv = lax.rsqrt(deg_ref[0] + deg_ref[1] + 1.0)
    hs = h_ref[...] * dinv[:, :1]
    xn_ref[0] = hs[:, :128]
    xn_ref[1] = hs[:, 128:]
    dinv_ref[...] = dinv


def _scale(h1, deg):
    n, h = h1.shape
    g = n // R
    return pl.pallas_call(
        _scale_body,
        grid=(g,),
        in_specs=[
            pl.BlockSpec((R, h), lambda i: (i, 0)),
            pl.BlockSpec((NC, R, L), lambda i: (0, i, 0)),
        ],
        out_specs=[
            pl.BlockSpec((NC, R, h // 2), lambda i: (0, i, 0)),
            pl.BlockSpec((R, L), lambda i: (i, 0)),
        ],
        out_shape=[
            jax.ShapeDtypeStruct((NC, n, h // 2), jnp.float32),
            jax.ShapeDtypeStruct((n, L), jnp.float32),
        ],
    )(h1, deg)


def _mm2_body(agg_ref, dinv_ref, b_ref, w_ref, xn_ref):
    dinv = dinv_ref[:, :1]
    a = jnp.concatenate([agg_ref[0], agg_ref[1]], axis=1)
    a = jnp.maximum(a * dinv + b_ref[...], 0.0)
    hs = jnp.dot(a, w_ref[...], preferred_element_type=jnp.float32) * dinv
    xn_ref[0] = hs[:, :128]
    xn_ref[1] = hs[:, 128:]


def _mm2(agg, dinv, b, w):
    n = agg.shape[1]
    h = w.shape[0]
    h2 = w.shape[1]
    g = n // R
    return pl.pallas_call(
        _mm2_body,
        grid=(g,),
        in_specs=[
            pl.BlockSpec((NC, R, h // 2), lambda i: (0, i, 0)),
            pl.BlockSpec((R, L), lambda i: (i, 0)),
            pl.BlockSpec((1, h), lambda i: (0, 0)),
            pl.BlockSpec((h, h2), lambda i: (0, 0)),
        ],
        out_specs=pl.BlockSpec((NC, R, h2 // 2), lambda i: (0, i, 0)),
        out_shape=jax.ShapeDtypeStruct((NC, n, h2 // 2), jnp.float32),
    )(agg, dinv, b, w)


def _head_body(agg_ref, dinv_ref, b_ref, wf1_ref, bf1_ref, wf2_ref, bf2_ref,
               out_ref):
    dinv = dinv_ref[:, :1]
    a = jnp.concatenate([agg_ref[0], agg_ref[1]], axis=1)
    a = jnp.maximum(a * dinv + b_ref[...], 0.0)
    f = jnp.maximum(
        jnp.dot(a, wf1_ref[...], preferred_element_type=jnp.float32)
        + bf1_ref[...], 0.0)
    o = (jnp.dot(f, wf2_ref[...], preferred_element_type=jnp.float32)
         + bf2_ref[...])
    m = jnp.max(o, axis=1, keepdims=True)
    z = o - m
    out_ref[...] = z - jnp.log(jnp.sum(jnp.exp(z), axis=1, keepdims=True))


def _head(agg, dinv, b, wf1, bf1, wf2, bf2):
    n = agg.shape[1]
    h = wf1.shape[0]
    o = wf2.shape[1]
    g = n // R
    return pl.pallas_call(
        _head_body,
        grid=(g,),
        in_specs=[
            pl.BlockSpec((NC, R, h // 2), lambda i: (0, i, 0)),
            pl.BlockSpec((R, L), lambda i: (i, 0)),
            pl.BlockSpec((1, h), lambda i: (0, 0)),
            pl.BlockSpec((h, h), lambda i: (0, 0)),
            pl.BlockSpec((1, h), lambda i: (0, 0)),
            pl.BlockSpec((h, o), lambda i: (0, 0)),
            pl.BlockSpec((1, o), lambda i: (0, 0)),
        ],
        out_specs=pl.BlockSpec((R, o), lambda i: (i, 0)),
        out_shape=jax.ShapeDtypeStruct((n, o), jnp.float32),
    )(agg, dinv, b, wf1, bf1, wf2, bf2)


def kernel(x, edge_index, W1, b1, W2, b2, Wf1, bf1, Wf2, bf2):
    n, d = x.shape
    e = edge_index.shape[1]
    src = edge_index[0].astype(jnp.int32)
    dst = edge_index[1].astype(jnp.int32)

    agg = _make_agg(n, e)
    h1 = _mm0(x, W1)
    deg = _make_deg(n, e)(dst)
    xn1, dinv = _scale(h1, deg)
    agg1 = agg(xn1.reshape(NC * n, 128), src, dst)
    xn2 = _mm2(agg1, dinv, b1.reshape(1, -1), W2)
    agg2 = agg(xn2.reshape(NC * n, 128), src, dst)
    return _head(agg2, dinv, b2.reshape(1, -1), Wf1,
                 bf1.reshape(1, -1), Wf2, bf2.reshape(1, -1))


# speedup vs baseline: 1.0585x; 1.0576x over previous
import jax
import jax.numpy as jnp
from jax import lax
from jax.experimental import pallas as pl
from jax.experimental.pallas import tpu as pltpu
from jax.experimental.pallas import tpu_sc as plsc

NC = 2
NS = 16
L = 16
B = 80
C = 10000
R = 400


def _sc_mesh():
    return plsc.VectorSubcoreMesh(
        core_axis_name="c", subcore_axis_name="s", num_cores=NC, num_subcores=NS
    )


def _make_deg(n, e):
    accrows = n + 8
    srows = (accrows // NS) // 8 * 8
    tail = accrows - NS * srows
    wrows = (n // NS) // 8 * 8
    wtail = n - NS * wrows
    e_tile = e // (NC * NS)
    e_pad = (e_tile + B - 1) // B * B
    nb = e_pad // B

    npadv = (e_pad - (e_tile // L) * L) // L

    def body(dst_hbm, init_hbm, deg_out, didxb, ones80, acc, sem):
        c = lax.axis_index("c")
        s = lax.axis_index("s")
        pltpu.sync_copy(init_hbm.at[pl.ds(0, srows)],
                        acc.at[pl.ds(s * srows, srows)])

        @pl.when(s == 0)
        def _():
            pltpu.sync_copy(init_hbm.at[pl.ds(0, tail)],
                            acc.at[pl.ds(NS * srows, tail)])

        pltpu.sync_copy(init_hbm.at[pl.ds(srows, B)], ones80)
        for j in range(npadv):
            didxb[pl.ds(e_pad - (j + 1) * L, L)] = jnp.full((L,), n, jnp.int32)
        pltpu.sync_copy(dst_hbm.at[pl.ds((c * NS + s) * e_tile, e_tile)],
                        didxb.at[pl.ds(0, e_tile)])
        plsc.subcore_barrier()

        def batch(b, _):
            pltpu.async_copy(ones80, acc.at[didxb.at[pl.ds(b * B, B)]],
                             sem, add=True)
            return 0

        lax.fori_loop(0, nb, batch, 0)

        def drain(b, _):
            pltpu.make_async_copy(ones80, acc.at[didxb.at[pl.ds(b * B, B)]],
                                  sem).wait()
            return 0

        lax.fori_loop(0, nb, drain, 0)
        plsc.subcore_barrier()
        pltpu.sync_copy(acc.at[pl.ds(s * wrows, wrows)],
                        deg_out.at[c, pl.ds(s * wrows, wrows)])

        @pl.when(s == 0)
        def _():
            pltpu.sync_copy(acc.at[pl.ds(NS * wrows, wtail)],
                            deg_out.at[c, pl.ds(NS * wrows, wtail)])

    def run(dst):
        init_hbm = jnp.concatenate(
            [jnp.zeros((srows, L), jnp.float32),
             jnp.ones((B, L), jnp.float32)])
        return pl.kernel(
            body,
            out_type=jax.ShapeDtypeStruct((NC, n, L), jnp.float32),
            mesh=_sc_mesh(),
            scratch_types=[
                pltpu.VMEM((e_pad,), jnp.int32),
                pltpu.VMEM((B, L), jnp.float32),
                pltpu.VMEM_SHARED((accrows, L), jnp.float32),
                pltpu.SemaphoreType.DMA,
            ],
        )(dst, init_hbm)

    return run


def _make_agg(n, e):
    d2 = 128
    srows = (n // NS) // 8 * 8
    tail = n - NS * srows
    e_tile = e // NS
    nch = e_tile // C
    cb = C // B

    def body(xn_hbm, src_hbm, dst_hbm, out_hbm,
             sidx, didx, r0, r1, acc, sg0, sg1, ss0, ss1):
        c = lax.axis_index("c")
        s = lax.axis_index("s")
        roff = c * n
        ebase = s * e_tile
        pltpu.sync_copy(xn_hbm.at[pl.ds(roff + s * srows, srows)],
                        acc.at[pl.ds(s * srows, srows)])

        @pl.when(s == 0)
        def _():
            pltpu.sync_copy(xn_hbm.at[pl.ds(roff + NS * srows, tail)],
                            acc.at[pl.ds(NS * srows, tail)])

        plsc.subcore_barrier()

        def fire_g(b, rows, sem):
            pltpu.async_copy(xn_hbm.at[sidx.at[pl.ds(b * B, B)]], rows, sem)

        def wait_g(b, rows, sem):
            pltpu.make_async_copy(xn_hbm.at[sidx.at[pl.ds(b * B, B)]],
                                  rows, sem).wait()

        def fire_s(b, rows, sem):
            pltpu.async_copy(rows, acc.at[didx.at[pl.ds(b * B, B)]], sem,
                             add=True)

        def wait_s(b, rows, sem):
            pltpu.make_async_copy(rows, acc.at[didx.at[pl.ds(b * B, B)]],
                                  sem).wait()

        for k in range(nch):
            pltpu.sync_copy(src_hbm.at[pl.ds(ebase + k * C, C)], sidx)
            pltpu.sync_copy(dst_hbm.at[pl.ds(ebase + k * C, C)], didx)

            def adjust(i, _):
                sidx[pl.ds(i * L, L)] = sidx[pl.ds(i * L, L)] + roff
                return 0

            lax.fori_loop(0, C // L, adjust, 0)

            fire_g(0, r0, sg0)

            def pair(p, _):
                b = 2 * p
                fire_g(b + 1, r1, sg1)
                wait_g(b, r0, sg0)
                fire_s(b, r0, ss0)
                wait_s(b, r0, ss0)
                fire_g(b + 2, r0, sg0)
                wait_g(b + 1, r1, sg1)
                fire_s(b + 1, r1, ss1)
                wait_s(b + 1, r1, ss1)
                return 0

            lax.fori_loop(0, (cb - 1) // 2, pair, 0)
            wait_g(cb - 1, r0, sg0)
            fire_s(cb - 1, r0, ss0)
            wait_s(cb - 1, r0, ss0)

        plsc.subcore_barrier()
        pltpu.sync_copy(acc.at[pl.ds(s * srows, srows)],
                        out_hbm.at[c, pl.ds(s * srows, srows)])

        @pl.when(s == 0)
        def _():
            pltpu.sync_copy(acc.at[pl.ds(NS * srows, tail)],
                            out_hbm.at[c, pl.ds(NS * srows, tail)])

    return pl.kernel(
        body,
        out_type=jax.ShapeDtypeStruct((NC, n, d2), jnp.float32),
        mesh=_sc_mesh(),
        scratch_types=[
            pltpu.VMEM((C,), jnp.int32),
            pltpu.VMEM((C,), jnp.int32),
            pltpu.VMEM((B, d2), jnp.float32),
            pltpu.VMEM((B, d2), jnp.float32),
            pltpu.VMEM_SHARED((n, d2), jnp.float32),
            pltpu.SemaphoreType.DMA,
            pltpu.SemaphoreType.DMA,
            pltpu.SemaphoreType.DMA,
            pltpu.SemaphoreType.DMA,
        ],
    )


def _mm1_body(x_ref, w_ref, deg_ref, xn_ref, dinv_ref):
    dinv = lax.rsqrt(deg_ref[0] + deg_ref[1] + 1.0)
    h = jnp.dot(x_ref[...], w_ref[...], preferred_element_type=jnp.float32)
    hs = h * dinv[:, :1]
    xn_ref[0] = hs[:, :128]
    xn_ref[1] = hs[:, 128:]
    dinv_ref[...] = dinv


def _mm1(x, w, deg):
    n, d = x.shape
    h = w.shape[1]
    g = n // R
    return pl.pallas_call(
        _mm1_body,
        grid=(g,),
        in_specs=[
            pl.BlockSpec((R, d), lambda i: (i, 0)),
            pl.BlockSpec((d, h), lambda i: (0, 0)),
            pl.BlockSpec((NC, R, L), lambda i: (0, i, 0)),
        ],
        out_specs=[
            pl.BlockSpec((NC, R, h // 2), lambda i: (0, i, 0)),
            pl.BlockSpec((R, L), lambda i: (i, 0)),
        ],
        out_shape=[
            jax.ShapeDtypeStruct((NC, n, h // 2), jnp.float32),
            jax.ShapeDtypeStruct((n, L), jnp.float32),
        ],
    )(x, w, deg)


def _mm2_body(agg_ref, dinv_ref, b_ref, w_ref, xn_ref):
    dinv = dinv_ref[:, :1]
    a = jnp.concatenate([agg_ref[0], agg_ref[1]], axis=1)
    a = jnp.maximum(a * dinv + b_ref[...], 0.0)
    hs = jnp.dot(a, w_ref[...], preferred_element_type=jnp.float32) * dinv
    xn_ref[0] = hs[:, :128]
    xn_ref[1] = hs[:, 128:]


def _mm2(agg, dinv, b, w):
    n = agg.shape[1]
    h = w.shape[0]
    h2 = w.shape[1]
    g = n // R
    return pl.pallas_call(
        _mm2_body,
        grid=(g,),
        in_specs=[
            pl.BlockSpec((NC, R, h // 2), lambda i: (0, i, 0)),
            pl.BlockSpec((R, L), lambda i: (i, 0)),
            pl.BlockSpec((1, h), lambda i: (0, 0)),
            pl.BlockSpec((h, h2), lambda i: (0, 0)),
        ],
        out_specs=pl.BlockSpec((NC, R, h2 // 2), lambda i: (0, i, 0)),
        out_shape=jax.ShapeDtypeStruct((NC, n, h2 // 2), jnp.float32),
    )(agg, dinv, b, w)


def _head_body(agg_ref, dinv_ref, b_ref, wf1_ref, bf1_ref, wf2_ref, bf2_ref,
               out_ref):
    dinv = dinv_ref[:, :1]
    a = jnp.concatenate([agg_ref[0], agg_ref[1]], axis=1)
    a = jnp.maximum(a * dinv + b_ref[...], 0.0)
    f = jnp.maximum(
        jnp.dot(a, wf1_ref[...], preferred_element_type=jnp.float32)
        + bf1_ref[...], 0.0)
    o = (jnp.dot(f, wf2_ref[...], preferred_element_type=jnp.float32)
         + bf2_ref[...])
    m = jnp.max(o, axis=1, keepdims=True)
    z = o - m
    out_ref[...] = z - jnp.log(jnp.sum(jnp.exp(z), axis=1, keepdims=True))


def _head(agg, dinv, b, wf1, bf1, wf2, bf2):
    n = agg.shape[1]
    h = wf1.shape[0]
    o = wf2.shape[1]
    g = n // R
    return pl.pallas_call(
        _head_body,
        grid=(g,),
        in_specs=[
            pl.BlockSpec((NC, R, h // 2), lambda i: (0, i, 0)),
            pl.BlockSpec((R, L), lambda i: (i, 0)),
            pl.BlockSpec((1, h), lambda i: (0, 0)),
            pl.BlockSpec((h, h), lambda i: (0, 0)),
            pl.BlockSpec((1, h), lambda i: (0, 0)),
            pl.BlockSpec((h, o), lambda i: (0, 0)),
            pl.BlockSpec((1, o), lambda i: (0, 0)),
        ],
        out_specs=pl.BlockSpec((R, o), lambda i: (i, 0)),
        out_shape=jax.ShapeDtypeStruct((n, o), jnp.float32),
    )(agg, dinv, b, wf1, bf1, wf2, bf2)


def kernel(x, edge_index, W1, b1, W2, b2, Wf1, bf1, Wf2, bf2):
    n, d = x.shape
    e = edge_index.shape[1]
    src = edge_index[0].astype(jnp.int32)
    dst = edge_index[1].astype(jnp.int32)

    agg = _make_agg(n, e)
    deg = _make_deg(n, e)(dst)
    xn1, dinv = _mm1(x, W1, deg)
    agg1 = agg(xn1.reshape(NC * n, 128), src, dst)
    xn2 = _mm2(agg1, dinv, b1.reshape(1, -1), W2)
    agg2 = agg(xn2.reshape(NC * n, 128), src, dst)
    return _head(agg2, dinv, b2.reshape(1, -1), Wf1,
                 bf1.reshape(1, -1), Wf2, bf2.reshape(1, -1))


# speedup vs baseline: 1.0915x; 1.0312x over previous
import jax
import jax.numpy as jnp
from jax import lax
from jax.experimental import pallas as pl
from jax.experimental.pallas import tpu as pltpu
from jax.experimental.pallas import tpu_sc as plsc

NC = 2
NS = 16
L = 16
B = 80
C = 10000
R = 400


def _sc_mesh():
    return plsc.VectorSubcoreMesh(
        core_axis_name="c", subcore_axis_name="s", num_cores=NC, num_subcores=NS
    )


def _make_deg(n, e):
    accrows = n + 8
    srows = (accrows // NS) // 8 * 8
    tail = accrows - NS * srows
    wrows = (n // NS) // 8 * 8
    wtail = n - NS * wrows
    e_tile = e // (NC * NS)
    e_pad = (e_tile + B - 1) // B * B
    nb = e_pad // B

    npadv = (e_pad - (e_tile // L) * L) // L

    def body(dst_hbm, init_hbm, deg_out, didxb, ones80, acc, sem):
        c = lax.axis_index("c")
        s = lax.axis_index("s")
        pltpu.sync_copy(init_hbm.at[pl.ds(0, srows)],
                        acc.at[pl.ds(s * srows, srows)])

        @pl.when(s == 0)
        def _():
            pltpu.sync_copy(init_hbm.at[pl.ds(0, tail)],
                            acc.at[pl.ds(NS * srows, tail)])

        pltpu.sync_copy(init_hbm.at[pl.ds(srows, B)], ones80)
        for j in range(npadv):
            didxb[pl.ds(e_pad - (j + 1) * L, L)] = jnp.full((L,), n, jnp.int32)
        pltpu.sync_copy(dst_hbm.at[pl.ds((c * NS + s) * e_tile, e_tile)],
                        didxb.at[pl.ds(0, e_tile)])
        plsc.subcore_barrier()

        def batch(b, _):
            pltpu.async_copy(ones80, acc.at[didxb.at[pl.ds(b * B, B)]],
                             sem, add=True)
            return 0

        lax.fori_loop(0, nb, batch, 0)

        def drain(b, _):
            pltpu.make_async_copy(ones80, acc.at[didxb.at[pl.ds(b * B, B)]],
                                  sem).wait()
            return 0

        lax.fori_loop(0, nb, drain, 0)
        plsc.subcore_barrier()
        pltpu.sync_copy(acc.at[pl.ds(s * wrows, wrows)],
                        deg_out.at[c, pl.ds(s * wrows, wrows)])

        @pl.when(s == 0)
        def _():
            pltpu.sync_copy(acc.at[pl.ds(NS * wrows, wtail)],
                            deg_out.at[c, pl.ds(NS * wrows, wtail)])

    def run(dst):
        init_hbm = jnp.concatenate(
            [jnp.zeros((srows, L), jnp.float32),
             jnp.ones((B, L), jnp.float32)])
        return pl.kernel(
            body,
            out_type=jax.ShapeDtypeStruct((NC, n, L), jnp.float32),
            mesh=_sc_mesh(),
            scratch_types=[
                pltpu.VMEM((e_pad,), jnp.int32),
                pltpu.VMEM((B, L), jnp.float32),
                pltpu.VMEM_SHARED((accrows, L), jnp.float32),
                pltpu.SemaphoreType.DMA,
            ],
        )(dst, init_hbm)

    return run


def _make_agg(n, e):
    d2 = 128
    srows = (n // NS) // 8 * 8
    tail = n - NS * srows
    e_tile = e // NS
    cb = C // B

    def body(xn_hbm, src_hbm, dst_hbm, out_hbm,
             sidx, didx, r0, r1, acc, sg0, sg1, ss0, ss1):
        c = lax.axis_index("c")
        s = lax.axis_index("s")
        ebase = s * e_tile
        pltpu.async_copy(src_hbm.at[pl.ds(c * e + ebase, C)], sidx, sg0)
        pltpu.async_copy(dst_hbm.at[pl.ds(ebase, C)], didx, sg1)
        pltpu.sync_copy(xn_hbm.at[pl.ds(c * n + s * srows, srows)],
                        acc.at[pl.ds(s * srows, srows)])

        @pl.when(s == 0)
        def _():
            pltpu.sync_copy(xn_hbm.at[pl.ds(c * n + NS * srows, tail)],
                            acc.at[pl.ds(NS * srows, tail)])

        pltpu.make_async_copy(src_hbm.at[pl.ds(c * e + ebase, C)],
                              sidx, sg0).wait()
        pltpu.make_async_copy(dst_hbm.at[pl.ds(ebase, C)],
                              didx, sg1).wait()
        plsc.subcore_barrier()

        def fire_g(b, rows, sem):
            pltpu.async_copy(xn_hbm.at[sidx.at[pl.ds(b * B, B)]], rows, sem)

        def wait_g(b, rows, sem):
            pltpu.make_async_copy(xn_hbm.at[sidx.at[pl.ds(b * B, B)]],
                                  rows, sem).wait()

        def fire_s(b, rows, sem):
            pltpu.async_copy(rows, acc.at[didx.at[pl.ds(b * B, B)]], sem,
                             add=True)

        def wait_s(b, rows, sem):
            pltpu.make_async_copy(rows, acc.at[didx.at[pl.ds(b * B, B)]],
                                  sem).wait()

        fire_g(0, r0, sg0)

        def pair(p, _):
            b = 2 * p
            fire_g(b + 1, r1, sg1)
            wait_g(b, r0, sg0)
            fire_s(b, r0, ss0)
            wait_s(b, r0, ss0)
            fire_g(b + 2, r0, sg0)
            wait_g(b + 1, r1, sg1)
            fire_s(b + 1, r1, ss1)
            wait_s(b + 1, r1, ss1)
            return 0

        lax.fori_loop(0, (cb - 1) // 2, pair, 0)
        wait_g(cb - 1, r0, sg0)
        fire_s(cb - 1, r0, ss0)
        wait_s(cb - 1, r0, ss0)

        plsc.subcore_barrier()
        pltpu.sync_copy(acc.at[pl.ds(s * srows, srows)],
                        out_hbm.at[c, pl.ds(s * srows, srows)])

        @pl.when(s == 0)
        def _():
            pltpu.sync_copy(acc.at[pl.ds(NS * srows, tail)],
                            out_hbm.at[c, pl.ds(NS * srows, tail)])

    return pl.kernel(
        body,
        out_type=jax.ShapeDtypeStruct((NC, n, d2), jnp.float32),
        mesh=_sc_mesh(),
        scratch_types=[
            pltpu.VMEM((C,), jnp.int32),
            pltpu.VMEM((C,), jnp.int32),
            pltpu.VMEM((B, d2), jnp.float32),
            pltpu.VMEM((B, d2), jnp.float32),
            pltpu.VMEM_SHARED((n, d2), jnp.float32),
            pltpu.SemaphoreType.DMA,
            pltpu.SemaphoreType.DMA,
            pltpu.SemaphoreType.DMA,
            pltpu.SemaphoreType.DMA,
        ],
    )


def _mm1_body(x_ref, w_ref, deg_ref, xn_ref, dinv_ref):
    dinv = lax.rsqrt(deg_ref[0] + deg_ref[1] + 1.0)
    h = jnp.dot(x_ref[...], w_ref[...], preferred_element_type=jnp.float32)
    hs = h * dinv[:, :1]
    xn_ref[0] = hs[:, :128]
    xn_ref[1] = hs[:, 128:]
    dinv_ref[...] = dinv


def _mm1(x, w, deg):
    n, d = x.shape
    h = w.shape[1]
    g = n // R
    return pl.pallas_call(
        _mm1_body,
        grid=(g,),
        in_specs=[
            pl.BlockSpec((R, d), lambda i: (i, 0)),
            pl.BlockSpec((d, h), lambda i: (0, 0)),
            pl.BlockSpec((NC, R, L), lambda i: (0, i, 0)),
        ],
        out_specs=[
            pl.BlockSpec((NC, R, h // 2), lambda i: (0, i, 0)),
            pl.BlockSpec((R, L), lambda i: (i, 0)),
        ],
        out_shape=[
            jax.ShapeDtypeStruct((NC, n, h // 2), jnp.float32),
            jax.ShapeDtypeStruct((n, L), jnp.float32),
        ],
    )(x, w, deg)


def _mm2_body(agg_ref, dinv_ref, b_ref, w_ref, xn_ref):
    dinv = dinv_ref[:, :1]
    a = jnp.concatenate([agg_ref[0], agg_ref[1]], axis=1)
    a = jnp.maximum(a * dinv + b_ref[...], 0.0)
    hs = jnp.dot(a, w_ref[...], preferred_element_type=jnp.float32) * dinv
    xn_ref[0] = hs[:, :128]
    xn_ref[1] = hs[:, 128:]


def _mm2(agg, dinv, b, w):
    n = agg.shape[1]
    h = w.shape[0]
    h2 = w.shape[1]
    g = n // R
    return pl.pallas_call(
        _mm2_body,
        grid=(g,),
        in_specs=[
            pl.BlockSpec((NC, R, h // 2), lambda i: (0, i, 0)),
            pl.BlockSpec((R, L), lambda i: (i, 0)),
            pl.BlockSpec((1, h), lambda i: (0, 0)),
            pl.BlockSpec((h, h2), lambda i: (0, 0)),
        ],
        out_specs=pl.BlockSpec((NC, R, h2 // 2), lambda i: (0, i, 0)),
        out_shape=jax.ShapeDtypeStruct((NC, n, h2 // 2), jnp.float32),
    )(agg, dinv, b, w)


def _head_body(agg_ref, dinv_ref, b_ref, wf1_ref, bf1_ref, wf2_ref, bf2_ref,
               out_ref):
    dinv = dinv_ref[:, :1]
    a = jnp.concatenate([agg_ref[0], agg_ref[1]], axis=1)
    a = jnp.maximum(a * dinv + b_ref[...], 0.0)
    f = jnp.maximum(
        jnp.dot(a, wf1_ref[...], preferred_element_type=jnp.float32)
        + bf1_ref[...], 0.0)
    o = (jnp.dot(f, wf2_ref[...], preferred_element_type=jnp.float32)
         + bf2_ref[...])
    m = jnp.max(o, axis=1, keepdims=True)
    z = o - m
    out_ref[...] = z - jnp.log(jnp.sum(jnp.exp(z), axis=1, keepdims=True))


def _head(agg, dinv, b, wf1, bf1, wf2, bf2):
    n = agg.shape[1]
    h = wf1.shape[0]
    o = wf2.shape[1]
    g = n // R
    return pl.pallas_call(
        _head_body,
        grid=(g,),
        in_specs=[
            pl.BlockSpec((NC, R, h // 2), lambda i: (0, i, 0)),
            pl.BlockSpec((R, L), lambda i: (i, 0)),
            pl.BlockSpec((1, h), lambda i: (0, 0)),
            pl.BlockSpec((h, h), lambda i: (0, 0)),
            pl.BlockSpec((1, h), lambda i: (0, 0)),
            pl.BlockSpec((h, o), lambda i: (0, 0)),
            pl.BlockSpec((1, o), lambda i: (0, 0)),
        ],
        out_specs=pl.BlockSpec((R, o), lambda i: (i, 0)),
        out_shape=jax.ShapeDtypeStruct((n, o), jnp.float32),
    )(agg, dinv, b, wf1, bf1, wf2, bf2)


def kernel(x, edge_index, W1, b1, W2, b2, Wf1, bf1, Wf2, bf2):
    n, d = x.shape
    e = edge_index.shape[1]
    src = edge_index[0].astype(jnp.int32)
    dst = edge_index[1].astype(jnp.int32)
    src2 = jnp.concatenate([src, src + jnp.int32(n)])

    agg = _make_agg(n, e)
    deg = _make_deg(n, e)(dst)
    xn1, dinv = _mm1(x, W1, deg)
    agg1 = agg(xn1.reshape(NC * n, 128), src2, dst)
    xn2 = _mm2(agg1, dinv, b1.reshape(1, -1), W2)
    agg2 = agg(xn2.reshape(NC * n, 128), src2, dst)
    return _head(agg2, dinv, b2.reshape(1, -1), Wf1,
                 bf1.reshape(1, -1), Wf2, bf2.reshape(1, -1))
